# baseline (speedup 1.0000x reference)
import functools
import jax
import jax.numpy as jnp
from jax import lax
from jax.experimental import pallas as pl
from jax.experimental.pallas import tpu as pltpu
from jax.experimental.pallas import tpu_sc as plsc

N = 10000
NP = 10240
BGR = 64
EE = 170000
EE_P = 172032
CH = 128
EPW1 = 5376
NCH1 = 42
NT1 = 10496
CHB = 2048
NCHB = 84
RPW = 320
DUM = RPW

_mesh = plsc.VectorSubcoreMesh(core_axis_name="c", subcore_axis_name="s")


@functools.partial(
    pl.kernel, mesh=_mesh,
    out_type=jax.ShapeDtypeStruct((2 * NT1,), jnp.float32),
    scratch_types=[
        pltpu.VMEM((CH,), jnp.int32),
        pltpu.VMEM((CH,), jnp.float32),
        pltpu.VMEM((656,), jnp.float32),
        pltpu.VMEM_SHARED((NT1,), jnp.float32),
    ],
)
def _k_deg(dstp_hbm, out_hbm, dst_v, val_v, zb_v, tbl_sh):
    c = lax.axis_index("c")
    s = lax.axis_index("s")
    z16 = jnp.zeros((16,), jnp.float32)
    for i in range(41):
        zb_v[pl.ds(i * 16, 16)] = z16
    for i in range(CH // 16):
        val_v[pl.ds(i * 16, 16)] = z16 + 1.0
    pltpu.sync_copy(zb_v, tbl_sh.at[pl.ds(s * 656, 656)])
    plsc.subcore_barrier()

    def chunk(k, carry):
        base = (s * 2 + c) * EPW1 + k * CH
        pltpu.sync_copy(dstp_hbm.at[pl.ds(base, CH)], dst_v)
        pltpu.sync_copy(val_v, tbl_sh.at[dst_v], add=True)
        return carry

    lax.fori_loop(0, NCH1, chunk, 0)
    plsc.subcore_barrier()
    pltpu.sync_copy(tbl_sh.at[pl.ds(s * 656, 656)], zb_v)
    pltpu.sync_copy(zb_v, out_hbm.at[pl.ds(c * NT1 + s * 656, 656)])


_I16 = None


def _iota16():
    return lax.broadcasted_iota(jnp.int32, (16,), 0)


@functools.partial(
    pl.kernel, mesh=_mesh,
    compiler_params=pltpu.CompilerParams(use_tc_tiling_on_sc=False, needs_layout_passes=False),
    out_type=jax.ShapeDtypeStruct((NP * 256,), jnp.float32),
    scratch_types=[
        pltpu.VMEM((CHB,), jnp.int32),
        pltpu.VMEM((CHB,), jnp.int32),
        pltpu.VMEM((CHB + 64,), jnp.int32),
        pltpu.VMEM((CHB + 64,), jnp.int32),
        pltpu.VMEM((64,), jnp.int32),
        pltpu.VMEM((64,), jnp.int32),
        pltpu.VMEM((64,), jnp.int32),
        pltpu.VMEM((64,), jnp.float32),
        pltpu.VMEM((64,), jnp.float32),
        pltpu.VMEM((64, 256), jnp.float32),
        pltpu.VMEM((328 * 256,), jnp.float32),
        pltpu.SemaphoreType.DMA,
    ],
)
def _k_propx(srcp_hbm, dstp_hbm, dinv_hbm, h_hbm, out_hbm,
             src_v, dst_v, st_src, st_ldst,
             idx64, ldst64, dstg64, dsb, ddb, rows_v, tbl, sem):
    c = lax.axis_index("c")
    s = lax.axis_index("s")
    w = s * 2 + c
    lo = w * RPW
    z16 = jnp.zeros((16,), jnp.float32)
    it16 = _iota16()

    def zr(i, carry):
        tbl[pl.ds(i * 16, 16)] = z16
        return carry
    lax.fori_loop(0, (328 * 256) // 16, zr, 0)

    def chunk(kc, carry):
        base = kc * CHB
        pltpu.sync_copy(srcp_hbm.at[pl.ds(base, CHB)], src_v)
        pltpu.sync_copy(dstp_hbm.at[pl.ds(base, CHB)], dst_v)

        def comp(i, off):
            s16 = src_v[pl.ds(i * 16, 16)]
            d16 = dst_v[pl.ds(i * 16, 16)]
            l16 = d16 - lo
            msk = (l16 >= 0) & (l16 < RPW)
            plsc.store_compressed(st_src.at[pl.ds(off, 16)], s16, mask=msk)
            plsc.store_compressed(st_ldst.at[pl.ds(off, 16)], l16, mask=msk)
            return off + jnp.sum(msk.astype(jnp.int32))

        off = lax.fori_loop(0, CHB // 16, comp, 0)
        for t in range(4):
            st_src[pl.ds(off + t * 16, 16)] = it16 * 0
            st_ldst[pl.ds(off + t * 16, 16)] = it16 * 0 + DUM
        nb = (off + 63) // 64

        def batch(bi, carry):
            sb = bi * 64
            for t in range(4):
                idx64[pl.ds(t * 16, 16)] = st_src[pl.ds(sb + t * 16, 16)]
                l16t = st_ldst[pl.ds(sb + t * 16, 16)]
                ldst64[pl.ds(t * 16, 16)] = l16t
                dstg64[pl.ds(t * 16, 16)] = jnp.minimum(l16t + lo, NP - 1)
            pltpu.async_copy(h_hbm.at[idx64], rows_v, sem).wait()
            pltpu.async_copy(dinv_hbm.at[idx64], dsb, sem).wait()
            pltpu.async_copy(dinv_hbm.at[dstg64], ddb, sem).wait()

            def edge(e, carry2):
                g = (e >> 4) * 16
                oh = it16 == (e & 15)
                l_e = jnp.sum(jnp.where(oh, ldst64[pl.ds(g, 16)], 0))
                ds_e = jnp.sum(jnp.where(oh, dsb[pl.ds(g, 16)], 0.0))
                dd_e = jnp.sum(jnp.where(oh, ddb[pl.ds(g, 16)], 0.0))
                n_e = ds_e * dd_e
                ib = l_e * 256
                for k in range(16):
                    v16 = rows_v[e, pl.ds(k * 16, 16)] * n_e
                    plsc.addupdate(tbl.at[pl.ds(ib + k * 16, 16)], v16)
                return carry2

            lax.fori_loop(0, 64, edge, 0)
            return carry

        lax.fori_loop(0, nb, batch, 0)
        return carry

    lax.fori_loop(0, NCHB, chunk, 0)
    pltpu.sync_copy(tbl.at[pl.ds(0, RPW * 256)],
                    out_hbm.at[pl.ds(w * RPW * 256, RPW * 256)])


@functools.partial(
    pl.kernel, mesh=_mesh,
    compiler_params=pltpu.CompilerParams(use_tc_tiling_on_sc=False, needs_layout_passes=False),
    out_type=jax.ShapeDtypeStruct((NP,), jnp.float32),
    scratch_types=[
        pltpu.VMEM((CHB,), jnp.int32),
        pltpu.VMEM((CHB,), jnp.int32),
        pltpu.VMEM((CHB + 64,), jnp.int32),
        pltpu.VMEM((CHB + 64,), jnp.int32),
        pltpu.VMEM((64,), jnp.int32),
        pltpu.VMEM((64,), jnp.int32),
        pltpu.VMEM((64,), jnp.int32),
        pltpu.VMEM((64,), jnp.float32),
        pltpu.VMEM((64,), jnp.float32),
        pltpu.VMEM((64,), jnp.float32),
        pltpu.VMEM((336,), jnp.float32),
        pltpu.SemaphoreType.DMA,
    ],
)
def _k_s1x(srcp_hbm, dstp_hbm, dinv_hbm, h_hbm, out_hbm,
           src_v, dst_v, st_src, st_ldst,
           idx64, ldst64, dstg64, dsb, ddb, hb, tbl, sem):
    c = lax.axis_index("c")
    s = lax.axis_index("s")
    w = s * 2 + c
    lo = w * RPW
    z16 = jnp.zeros((16,), jnp.float32)
    it16 = _iota16()
    oh0 = it16 == 0
    for i in range(336 // 16):
        tbl[pl.ds(i * 16, 16)] = z16

    def chunk(kc, carry):
        base = kc * CHB
        pltpu.sync_copy(srcp_hbm.at[pl.ds(base, CHB)], src_v)
        pltpu.sync_copy(dstp_hbm.at[pl.ds(base, CHB)], dst_v)

        def comp(i, off):
            s16 = src_v[pl.ds(i * 16, 16)]
            d16 = dst_v[pl.ds(i * 16, 16)]
            l16 = d16 - lo
            msk = (l16 >= 0) & (l16 < RPW)
            plsc.store_compressed(st_src.at[pl.ds(off, 16)], s16, mask=msk)
            plsc.store_compressed(st_ldst.at[pl.ds(off, 16)], l16, mask=msk)
            return off + jnp.sum(msk.astype(jnp.int32))

        off = lax.fori_loop(0, CHB // 16, comp, 0)
        for t in range(4):
            st_src[pl.ds(off + t * 16, 16)] = it16 * 0
            st_ldst[pl.ds(off + t * 16, 16)] = it16 * 0 + DUM
        nb = (off + 63) // 64

        def batch(bi, carry2):
            sb = bi * 64
            for t in range(4):
                idx64[pl.ds(t * 16, 16)] = st_src[pl.ds(sb + t * 16, 16)]
                l16t = st_ldst[pl.ds(sb + t * 16, 16)]
                ldst64[pl.ds(t * 16, 16)] = l16t
                dstg64[pl.ds(t * 16, 16)] = jnp.minimum(l16t + lo, NP - 1)
            pltpu.async_copy(dinv_hbm.at[idx64], dsb, sem).wait()
            pltpu.async_copy(dinv_hbm.at[dstg64], ddb, sem).wait()
            pltpu.async_copy(h_hbm.at[idx64], hb, sem).wait()

            def edge(e, carry3):
                g = (e >> 4) * 16
                oh = it16 == (e & 15)
                l_e = jnp.sum(jnp.where(oh, ldst64[pl.ds(g, 16)], 0))
                ds_e = jnp.sum(jnp.where(oh, dsb[pl.ds(g, 16)], 0.0))
                dd_e = jnp.sum(jnp.where(oh, ddb[pl.ds(g, 16)], 0.0))
                h_e = jnp.sum(jnp.where(oh, hb[pl.ds(g, 16)], 0.0))
                u_e = h_e * (ds_e * dd_e)
                v16 = jnp.where(oh0, u_e, 0.0)
                plsc.addupdate(tbl.at[pl.ds(l_e, 16)], v16)
                return carry3

            lax.fori_loop(0, 64, edge, 0)
            return carry2

        lax.fori_loop(0, nb, batch, 0)
        return carry

    lax.fori_loop(0, NCHB, chunk, 0)
    pltpu.sync_copy(tbl.at[pl.ds(0, RPW)], out_hbm.at[pl.ds(w * RPW, RPW)])


@functools.partial(
    pl.kernel, mesh=_mesh,
    out_type=(jax.ShapeDtypeStruct((2048, 256), jnp.float32),
              jax.ShapeDtypeStruct((2048, 256), jnp.float32),
              jax.ShapeDtypeStruct((2048, 256), jnp.float32)),
    scratch_types=[
        pltpu.VMEM((64,), jnp.int32),
        pltpu.VMEM((64, 256), jnp.float32),
        pltpu.VMEM((64, 256), jnp.float32),
        pltpu.VMEM((64, 256), jnp.float32),
        pltpu.SemaphoreType.DMA,
    ],
)
def _k_gather(idx_hbm, x1_hbm, x2_hbm, x3_hbm, g1_hbm, g2_hbm, g3_hbm,
              idx_v, r1_v, r2_v, r3_v, sem):
    c = lax.axis_index("c")
    s = lax.axis_index("s")
    base = (s * 2 + c) * 64
    pltpu.sync_copy(idx_hbm.at[pl.ds(base, 64)], idx_v)
    pltpu.async_copy(x1_hbm.at[idx_v], r1_v, sem).wait()
    pltpu.async_copy(x2_hbm.at[idx_v], r2_v, sem).wait()
    pltpu.async_copy(x3_hbm.at[idx_v], r3_v, sem).wait()
    pltpu.sync_copy(r1_v, g1_hbm.at[pl.ds(base, 64)])
    pltpu.sync_copy(r2_v, g2_hbm.at[pl.ds(base, 64)])
    pltpu.sync_copy(r3_v, g3_hbm.at[pl.ds(base, 64)])


def _mm_body(x_ref, w_ref, o_ref):
    o_ref[...] = jnp.dot(x_ref[...], w_ref[...],
                         preferred_element_type=jnp.float32)


def _tc_mm(x, w):
    wout = w.shape[1]
    return pl.pallas_call(
        _mm_body,
        grid=(NP // 256,),
        in_specs=[
            pl.BlockSpec((256, 256), lambda i: (i, 0)),
            pl.BlockSpec((256, wout), lambda i: (0, 0)),
        ],
        out_specs=pl.BlockSpec((256, wout), lambda i: (i, 0)),
        out_shape=jax.ShapeDtypeStruct((NP, wout), jnp.float32),
    )(x, w)


def _sel_body(x4_ref, bt_ref, selidx_ref, selval_ref, valid_ref):
    x4 = x4_ref[...]
    biota = lax.broadcasted_iota(jnp.int32, (BGR, NP), 0)
    citer = lax.broadcasted_iota(jnp.int32, (BGR, NP), 1)
    bt = jnp.broadcast_to(bt_ref[...], (BGR, NP))
    vals = jnp.where(bt == biota, jnp.broadcast_to(x4, (BGR, NP)), -1e30)
    for j in range(32):
        m = jnp.max(vals, axis=1, keepdims=True)
        selc = jnp.min(jnp.where(vals >= m, citer, NP + 5), axis=1,
                       keepdims=True)
        valid = m > -1e29
        selidx_ref[:, pl.ds(j, 1)] = jnp.where(valid, selc, 0)
        selval_ref[:, pl.ds(j, 1)] = jnp.where(valid, m, 0.0)
        valid_ref[:, pl.ds(j, 1)] = valid.astype(jnp.float32)
        vals = jnp.where((citer == selc) & valid, -1e30, vals)


def _tc_sel(x4r, batchr):
    return pl.pallas_call(
        _sel_body,
        out_shape=[
            jax.ShapeDtypeStruct((BGR, 32), jnp.int32),
            jax.ShapeDtypeStruct((BGR, 32), jnp.float32),
            jax.ShapeDtypeStruct((BGR, 32), jnp.float32),
        ],
    )(x4r, batchr)


def _tail_body(g1_ref, g2_ref, g3_ref, vld_ref, sval_ref, a1_ref, a2_ref,
               a3_ref, w4_ref, bc1_ref, wc2_ref, bc2_ref, wm1_ref, bm1_ref,
               wm2_ref, bm2_ref, out_ref):
    vld = vld_ref[...]
    f = (jnp.dot(g1_ref[...] * vld, a1_ref[...], preferred_element_type=jnp.float32)
         + jnp.dot(g2_ref[...] * vld, a2_ref[...], preferred_element_type=jnp.float32)
         + jnp.dot(g3_ref[...] * vld, a3_ref[...], preferred_element_type=jnp.float32)
         + sval_ref[...] * w4_ref[...] + bc1_ref[...])
    f = jnp.maximum(f, 0.0)[:1920]
    m = jnp.concatenate(
        [jnp.maximum(f[(2 * t) * 64:(2 * t + 1) * 64],
                     f[(2 * t + 1) * 64:(2 * t + 2) * 64]) for t in range(15)],
        axis=0)
    h = jnp.zeros((BGR, 128), jnp.float32)
    for u in range(11):
        v = jnp.concatenate(
            [m[(u + r) * 64:(u + r + 1) * 64] for r in range(5)], axis=1)
        g = jnp.maximum(jnp.dot(v, wc2_ref[...],
                                preferred_element_type=jnp.float32)
                        + bc2_ref[...], 0.0)
        h = h + jnp.dot(g, wm1_ref[pl.ds(u * 32, 32), :],
                        preferred_element_type=jnp.float32)
    h = jnp.maximum(h + bm1_ref[...], 0.0)
    out_ref[...] = jnp.dot(h, wm2_ref[...],
                           preferred_element_type=jnp.float32) + bm2_ref[...]


def _tc_tail(g1, g2, g3, vldf, svalf, a1, a2, a3, w4r, bc1r, wc2m, bc2r,
             wm1r, bm1r, wm2p, bm2r):
    return pl.pallas_call(
        _tail_body,
        out_shape=jax.ShapeDtypeStruct((BGR, 8), jnp.float32),
    )(g1, g2, g3, vldf, svalf, a1, a2, a3, w4r, bc1r, wc2m, bc2r,
      wm1r, bm1r, wm2p, bm2r)


@jax.jit
def kernel(x, edge_index, batch, W0, b0, W1, b1, W2, b2, W3, b3,
           Wc1, bc1, Wc2, bc2, Wm1, bm1, Wm2, bm2):
    loop = jnp.arange(N, dtype=jnp.int32)
    padn = EE_P - EE
    srcp = jnp.concatenate([edge_index[0].astype(jnp.int32), loop,
                            jnp.full((padn,), N, jnp.int32)])
    dstp = jnp.concatenate([edge_index[1].astype(jnp.int32), loop,
                            jnp.full((padn,), NP, jnp.int32)])
    xp = jnp.pad(x, ((0, NP - N), (0, 0)))

    degp = _k_deg(dstp)
    deg = (degp[:NT1] + degp[NT1:])[:NP]
    dinv = jnp.where(deg > 0, 1.0 / jnp.sqrt(deg), 0.0)

    h0 = _tc_mm(xp, W0)
    a0 = _k_propx(srcp, dstp, dinv, h0).reshape(NP, 256)
    x1 = jnp.tanh(a0 + b0)
    h1 = _tc_mm(x1, W1)
    a1 = _k_propx(srcp, dstp, dinv, h1).reshape(NP, 256)
    x2 = jnp.tanh(a1 + b1)
    h2 = _tc_mm(x2, W2)
    a2 = _k_propx(srcp, dstp, dinv, h2).reshape(NP, 256)
    x3 = jnp.tanh(a2 + b2)
    w3p = jnp.pad(W3, ((0, 0), (0, 127)))
    h3 = _tc_mm(x3, w3p)[:, 0]
    a3 = _k_s1x(srcp, dstp, dinv, h3)
    x4 = jnp.tanh(a3 + b3[0])

    batchp = jnp.pad(batch.astype(jnp.int32), (0, NP - N),
                     constant_values=BGR)
    selidx, selval, validf = _tc_sel(x4.reshape(1, NP),
                                     batchp.reshape(1, NP))

    idx_flat = jnp.transpose(selidx).reshape(-1)
    val_flat = jnp.transpose(selval).reshape(-1, 1)
    vld_flat = jnp.transpose(validf).reshape(-1, 1)

    g1, g2, g3 = _k_gather(idx_flat, x1, x2, x3)

    a1w = jnp.transpose(Wc1[:, 0:256])
    a2w = jnp.transpose(Wc1[:, 256:512])
    a3w = jnp.transpose(Wc1[:, 512:768])
    w4r = Wc1[:, 768].reshape(1, 16)
    wc2m = jnp.transpose(Wc2, (2, 1, 0)).reshape(80, 32)
    wm1r = jnp.transpose(Wm1.reshape(32, 11, 128), (1, 0, 2)).reshape(352, 128)
    wm2p = jnp.pad(Wm2, ((0, 0), (0, 7)))
    bm2r = jnp.pad(bm2, (0, 7)).reshape(1, 8)

    out = _tc_tail(g1, g2, g3, vld_flat, val_flat, a1w, a2w, a3w, w4r,
                   bc1.reshape(1, 16), wc2m, bc2.reshape(1, 32),
                   wm1r, bm1.reshape(1, 128), wm2p, bm2r)
    return out[:, :1]
